# Initial kernel scaffold; baseline (speedup 1.0000x reference)
import jax
import jax.numpy as jnp
from jax.experimental import pallas as pl


def kernel(x, edge_index, W_att, a_att, W_hid, a_hid, W_out, a_out):
    raise NotImplementedError("write your pallas kernel here")


# speedup vs baseline: 34.3084x; 34.3084x over previous
import functools

import jax
import jax.numpy as jnp
from jax import lax
from jax.experimental import pallas as pl
from jax.experimental.pallas import tpu as pltpu
from jax.experimental.pallas import tpu_sc as plsc

N = 10000
E = 320000
IN_F = 128
HID = 16
HEADS = 8
OUT_F = 64
ALPHA = 0.2
NEG = -1e30

NC = 2
NT = 16
NW = NC * NT
PT = E // NW
C = 80
NCH = PT // C
NP = 10240
RPT = NP // NT
ZR = 128

R = 1000


def _lrelu(v):
    return jnp.where(v > 0, v, ALPHA * v)


def _p1_body(x_ref, wcat_ref, adm_ref, asm_ref, g_ref, adt_ref, mx_ref):
    h = jnp.dot(x_ref[...], wcat_ref[...], preferred_element_type=jnp.float32)
    asrc = jnp.dot(h, asm_ref[...], preferred_element_type=jnp.float32)
    adst = jnp.dot(h, adm_ref[...], preferred_element_type=jnp.float32)
    g_ref[...] = jnp.concatenate(
        [h, asrc, jnp.full((h.shape[0], 8), NEG, jnp.float32)], axis=1)
    adt_ref[...] = jnp.concatenate(
        [adst, jnp.zeros((h.shape[0], 8), jnp.float32)], axis=1)
    i = pl.program_id(0)

    @pl.when(i == 0)
    def _():
        mx_ref[...] = jnp.full((2, 16), -jnp.inf, jnp.float32)

    cd = jnp.max(adst, axis=0)
    cs = jnp.max(asrc, axis=0)
    pad = jnp.full((8,), -jnp.inf, jnp.float32)
    mx_ref[0, :] = jnp.maximum(mx_ref[0, :], jnp.concatenate([cd, pad]))
    mx_ref[1, :] = jnp.maximum(mx_ref[1, :], jnp.concatenate([cs, pad]))

    @pl.when(i == pl.num_programs(0) - 1)
    def _():
        g8 = _lrelu(mx_ref[0, 0:8] + mx_ref[1, 0:8])
        mx_ref[0, :] = jnp.concatenate([g8, jnp.zeros((8,), jnp.float32)])


def _p23_body(w_ref, asm_ref, adm_ref, r8_ref, p0_ref, p1_ref,
              g_ref, adt_ref, mx_ref, *, relu):
    s = p0_ref[...] + p1_ref[...]
    den = jnp.dot(s[:, IN_F:IN_F + 8], r8_ref[...],
                  preferred_element_type=jnp.float32) + 1e-16
    x = s[:, 0:IN_F] / den
    if relu:
        x = jnp.maximum(x, 0.0)
    else:
        x = jnp.where(x > 0, x, jnp.exp(x) - 1.0)
    h = jnp.dot(x, w_ref[...], preferred_element_type=jnp.float32)
    asrc = jnp.dot(h, asm_ref[...], preferred_element_type=jnp.float32)
    adst = jnp.dot(h, adm_ref[...], preferred_element_type=jnp.float32)
    g_ref[...] = jnp.concatenate([h, asrc], axis=1)
    adt_ref[...] = adst
    i = pl.program_id(0)

    @pl.when(i == 0)
    def _():
        mx_ref[...] = jnp.full((2, 16), -jnp.inf, jnp.float32)

    mx_ref[0, :] = jnp.maximum(mx_ref[0, :], jnp.max(adst, axis=0))
    mx_ref[1, :] = jnp.maximum(mx_ref[1, :], jnp.max(asrc, axis=0))

    @pl.when(i == pl.num_programs(0) - 1)
    def _():
        mx_ref[0, :] = _lrelu(mx_ref[0, :] + mx_ref[1, :])


def _p4_body(rep_ref, p0_ref, p1_ref, o_ref):
    s = p0_ref[...] + p1_ref[...]
    den = jnp.dot(s[:, OUT_F:OUT_F + 16], rep_ref[...],
                  preferred_element_type=jnp.float32) + 1e-16
    logits = s[:, 0:OUT_F] / den
    m = jnp.max(logits, axis=1, keepdims=True)
    z = logits - m
    lse = jnp.log(jnp.sum(jnp.exp(z), axis=1, keepdims=True))
    o_ref[...] = z - lse


def _row_spec(width):
    return pl.BlockSpec((R, width), lambda i: (i, 0))


def _full_spec(a, b):
    return pl.BlockSpec((a, b), lambda i: (0, 0))


def _part_spec(j, width):
    return pl.BlockSpec((None, R, width), lambda i, j=j: (j, i, 0))


_MX_SPEC = pl.BlockSpec((2, 16), lambda i: (0, 0))


def _make_edge_kernel(FH, ROWW, n_heads):
    mesh = plsc.VectorSubcoreMesh(core_axis_name="c", subcore_axis_name="s")

    @functools.partial(
        pl.kernel,
        out_type=jax.ShapeDtypeStruct((NC, NP, ROWW), jnp.float32),
        mesh=mesh,
        compiler_params=pltpu.CompilerParams(use_tc_tiling_on_sc=False),
        scratch_types=[
            pltpu.VMEM_SHARED((NP, ROWW), jnp.float32),
            pltpu.VMEM((C,), jnp.int32),
            pltpu.VMEM((C,), jnp.int32),
            pltpu.VMEM((C, ROWW), jnp.float32),
            pltpu.VMEM((C, 16), jnp.float32),
            pltpu.VMEM((16,), jnp.float32),
            pltpu.VMEM((ZR, ROWW), jnp.float32),
            pltpu.SemaphoreType.DMA,
            pltpu.SemaphoreType.DMA,
        ],
    )
    def k(g_hbm, src_hbm, dst_hbm, adt_hbm, mx_hbm, out_hbm,
          acc, srcv, dstv, gbuf, adbuf, gmaxv, zbuf, sem_g, sem_a):
        cid = lax.axis_index("c")
        sid = lax.axis_index("s")
        wid = sid * NC + cid

        nzv = ROWW // 16

        def zb(i, _):
            r = i // nzv
            col = i % nzv
            zbuf[r, pl.ds(col * 16, 16)] = jnp.zeros((16,), jnp.float32)
            return 0

        lax.fori_loop(0, ZR * nzv, zb, 0)

        t0 = sid * RPT

        def zc(j, _):
            pltpu.sync_copy(zbuf, acc.at[pl.ds(t0 + j * ZR, ZR)])
            return 0

        lax.fori_loop(0, RPT // ZR, zc, 0)
        pltpu.sync_copy(mx_hbm.at[0], gmaxv)
        plsc.subcore_barrier()

        base_w = wid * PT
        gm = gmaxv[:]

        def chunk(i, _):
            b = base_w + i * C
            pltpu.sync_copy(src_hbm.at[pl.ds(b, C)], srcv)
            pltpu.sync_copy(dst_hbm.at[pl.ds(b, C)], dstv)
            cp1 = pltpu.async_copy(g_hbm.at[srcv], gbuf, sem_g)
            cp2 = pltpu.async_copy(adt_hbm.at[dstv], adbuf, sem_a)
            cp1.wait()
            cp2.wait()

            def edge(e, _2):
                ev = adbuf[e, :] + gbuf[e, pl.ds(FH, 16)]
                ev = jnp.where(ev > 0, ev, ALPHA * ev)
                w = jnp.exp(ev - gm)
                gbuf[e, pl.ds(FH, 16)] = w
                if n_heads == 1:
                    for kk in range(FH // 16):
                        gbuf[e, pl.ds(kk * 16, 16)] = (
                            gbuf[e, pl.ds(kk * 16, 16)] * w)
                else:
                    for hh in range(n_heads):
                        ws = w[hh]
                        gbuf[e, pl.ds(hh * 16, 16)] = (
                            gbuf[e, pl.ds(hh * 16, 16)] * ws)
                return 0

            lax.fori_loop(0, C, edge, 0)
            pltpu.sync_copy(gbuf, acc.at[dstv], add=True)
            return 0

        lax.fori_loop(0, NCH, chunk, 0)
        plsc.subcore_barrier()
        pltpu.sync_copy(acc.at[pl.ds(t0, RPT)],
                        out_hbm.at[cid, pl.ds(t0, RPT)])

    return k


@functools.lru_cache(maxsize=None)
def _edge_kernels():
    return (_make_edge_kernel(IN_F, 144, HEADS),
            _make_edge_kernel(IN_F, 144, 1),
            _make_edge_kernel(OUT_F, 80, 1))


def kernel(x, edge_index, W_att, a_att, W_hid, a_hid, W_out, a_out):
    f32 = jnp.float32
    x = x.astype(f32)
    src = edge_index[0]
    dst = edge_index[1]
    grid = (N // R,)

    Wcat = jnp.transpose(W_att, (1, 0, 2)).reshape(IN_F, HEADS * HID)
    eye8 = jnp.eye(HEADS, dtype=f32)
    Adst1 = (a_att[:, :HID][:, :, None] * eye8[:, None, :]).reshape(
        HEADS * HID, HEADS)
    Asrc1 = (a_att[:, HID:][:, :, None] * eye8[:, None, :]).reshape(
        HEADS * HID, HEADS)
    R8 = jnp.repeat(eye8, HID, axis=1)
    ones16 = jnp.ones((1, 16), f32)
    Adst2 = a_hid[:IN_F][:, None] * ones16
    Asrc2 = a_hid[IN_F:][:, None] * ones16
    Adst3 = a_out[:OUT_F][:, None] * ones16
    Asrc3 = a_out[OUT_F:][:, None] * ones16
    Rep = jnp.full((16, OUT_F), 1.0 / 16.0, f32)
    _sc1, _sc2, _sc3 = _edge_kernels()

    g1, adt1, mx1 = pl.pallas_call(
        _p1_body,
        grid=grid,
        in_specs=[_row_spec(IN_F), _full_spec(IN_F, IN_F),
                  _full_spec(IN_F, HEADS), _full_spec(IN_F, HEADS)],
        out_specs=[_row_spec(144), _row_spec(16), _MX_SPEC],
        out_shape=[jax.ShapeDtypeStruct((N, 144), f32),
                   jax.ShapeDtypeStruct((N, 16), f32),
                   jax.ShapeDtypeStruct((2, 16), f32)],
    )(x, Wcat, Adst1, Asrc1)

    part1 = _sc1(g1, src, dst, adt1, mx1)

    g2, adt2, mx2 = pl.pallas_call(
        functools.partial(_p23_body, relu=False),
        grid=grid,
        in_specs=[_full_spec(IN_F, IN_F), _full_spec(IN_F, 16),
                  _full_spec(IN_F, 16), _full_spec(8, IN_F),
                  _part_spec(0, 144), _part_spec(1, 144)],
        out_specs=[_row_spec(144), _row_spec(16), _MX_SPEC],
        out_shape=[jax.ShapeDtypeStruct((N, 144), f32),
                   jax.ShapeDtypeStruct((N, 16), f32),
                   jax.ShapeDtypeStruct((2, 16), f32)],
    )(W_hid, Asrc2, Adst2, R8, part1, part1)

    part2 = _sc2(g2, src, dst, adt2, mx2)

    g3, adt3, mx3 = pl.pallas_call(
        functools.partial(_p23_body, relu=True),
        grid=grid,
        in_specs=[_full_spec(IN_F, OUT_F), _full_spec(OUT_F, 16),
                  _full_spec(OUT_F, 16), _full_spec(8, IN_F),
                  _part_spec(0, 144), _part_spec(1, 144)],
        out_specs=[_row_spec(80), _row_spec(16), _MX_SPEC],
        out_shape=[jax.ShapeDtypeStruct((N, 80), f32),
                   jax.ShapeDtypeStruct((N, 16), f32),
                   jax.ShapeDtypeStruct((2, 16), f32)],
    )(W_out, Asrc3, Adst3, R8, part2, part2)

    part3 = _sc3(g3, src, dst, adt3, mx3)

    out = pl.pallas_call(
        _p4_body,
        grid=grid,
        in_specs=[_full_spec(16, OUT_F),
                  _part_spec(0, 80), _part_spec(1, 80)],
        out_specs=[_row_spec(OUT_F)],
        out_shape=[jax.ShapeDtypeStruct((N, OUT_F), f32)],
    )(Rep, part3, part3)[0]

    return out


# speedup vs baseline: 59.1107x; 1.7229x over previous
import functools

import jax
import jax.numpy as jnp
from jax import lax
from jax.experimental import pallas as pl
from jax.experimental.pallas import tpu as pltpu
from jax.experimental.pallas import tpu_sc as plsc

N = 10000
E = 320000
IN_F = 128
HID = 16
HEADS = 8
OUT_F = 64
ALPHA = 0.2
NEG = -1e30

NC = 2
NT = 16
NW = NC * NT
PT = E // NW
C = 80
NCH = PT // C
NP = 10240
RPT = NP // NT
ZR = 16
SEGCH = 25
NSEG = 5

R = 1000


def _lrelu(v):
    return jnp.where(v > 0, v, ALPHA * v)


def _p1_body(x_ref, wcat_ref, adm_ref, asm_ref, g_ref, adt_ref, mx_ref):
    h = jnp.dot(x_ref[...], wcat_ref[...], preferred_element_type=jnp.float32)
    asrc = jnp.dot(h, asm_ref[...], preferred_element_type=jnp.float32)
    adst = jnp.dot(h, adm_ref[...], preferred_element_type=jnp.float32)
    g_ref[...] = jnp.concatenate(
        [h, asrc, jnp.full((h.shape[0], 8), NEG, jnp.float32)], axis=1)
    adt_ref[...] = jnp.concatenate(
        [adst, jnp.zeros((h.shape[0], 8), jnp.float32)], axis=1)
    i = pl.program_id(0)

    @pl.when(i == 0)
    def _():
        mx_ref[...] = jnp.full((2, 16), -jnp.inf, jnp.float32)

    cd = jnp.max(adst, axis=0)
    cs = jnp.max(asrc, axis=0)
    pad = jnp.full((8,), -jnp.inf, jnp.float32)
    mx_ref[0, :] = jnp.maximum(mx_ref[0, :], jnp.concatenate([cd, pad]))
    mx_ref[1, :] = jnp.maximum(mx_ref[1, :], jnp.concatenate([cs, pad]))

    @pl.when(i == pl.num_programs(0) - 1)
    def _():
        g8 = _lrelu(mx_ref[0, 0:8] + mx_ref[1, 0:8])
        mx_ref[0, :] = jnp.concatenate([g8, jnp.zeros((8,), jnp.float32)])


def _p23_body(w_ref, asm_ref, adm_ref, r8_ref, p0_ref, p1_ref,
              g_ref, adt_ref, mx_ref, *, relu):
    s = p0_ref[...] + p1_ref[...]
    den = jnp.dot(s[:, IN_F:IN_F + 8], r8_ref[...],
                  preferred_element_type=jnp.float32) + 1e-16
    x = s[:, 0:IN_F] / den
    if relu:
        x = jnp.maximum(x, 0.0)
    else:
        x = jnp.where(x > 0, x, jnp.exp(x) - 1.0)
    h = jnp.dot(x, w_ref[...], preferred_element_type=jnp.float32)
    asrc = jnp.dot(h, asm_ref[...], preferred_element_type=jnp.float32)
    adst = jnp.dot(h, adm_ref[...], preferred_element_type=jnp.float32)
    g_ref[...] = jnp.concatenate([h, asrc], axis=1)
    adt_ref[...] = adst
    i = pl.program_id(0)

    @pl.when(i == 0)
    def _():
        mx_ref[...] = jnp.full((2, 16), -jnp.inf, jnp.float32)

    mx_ref[0, :] = jnp.maximum(mx_ref[0, :], jnp.max(adst, axis=0))
    mx_ref[1, :] = jnp.maximum(mx_ref[1, :], jnp.max(asrc, axis=0))

    @pl.when(i == pl.num_programs(0) - 1)
    def _():
        mx_ref[0, :] = _lrelu(mx_ref[0, :] + mx_ref[1, :])


def _p4_body(rep_ref, p0_ref, p1_ref, o_ref):
    s = p0_ref[...] + p1_ref[...]
    den = jnp.dot(s[:, OUT_F:OUT_F + 16], rep_ref[...],
                  preferred_element_type=jnp.float32) + 1e-16
    logits = s[:, 0:OUT_F] / den
    m = jnp.max(logits, axis=1, keepdims=True)
    z = logits - m
    lse = jnp.log(jnp.sum(jnp.exp(z), axis=1, keepdims=True))
    o_ref[...] = z - lse


def _row_spec(width):
    return pl.BlockSpec((R, width), lambda i: (i, 0))


def _full_spec(a, b):
    return pl.BlockSpec((a, b), lambda i: (0, 0))


def _part_spec(j, width):
    return pl.BlockSpec((None, R, width), lambda i, j=j: (j, i, 0))


_MX_SPEC = pl.BlockSpec((2, 16), lambda i: (0, 0))


def _make_edge_kernel(FH, ROWW, n_heads):
    mesh = plsc.VectorSubcoreMesh(core_axis_name="c", subcore_axis_name="s")

    @functools.partial(
        pl.kernel,
        out_type=jax.ShapeDtypeStruct((NC, NP, ROWW), jnp.float32),
        mesh=mesh,
        compiler_params=pltpu.CompilerParams(use_tc_tiling_on_sc=False),
        scratch_types=[
            pltpu.VMEM_SHARED((NP, ROWW), jnp.float32),
            pltpu.VMEM((SEGCH, C), jnp.int32),
            pltpu.VMEM((SEGCH, C), jnp.int32),
            pltpu.VMEM((SEGCH, C), jnp.int32),
            pltpu.VMEM((SEGCH, C), jnp.int32),
            pltpu.VMEM((C, ROWW), jnp.float32),
            pltpu.VMEM((C, ROWW), jnp.float32),
            pltpu.VMEM((C, 16), jnp.float32),
            pltpu.VMEM((C, 16), jnp.float32),
            pltpu.VMEM((16,), jnp.float32),
            pltpu.VMEM((ZR, ROWW), jnp.float32),
            pltpu.SemaphoreType.DMA,
            pltpu.SemaphoreType.DMA,
            pltpu.SemaphoreType.DMA,
            pltpu.SemaphoreType.DMA,
            pltpu.SemaphoreType.DMA,
            pltpu.SemaphoreType.DMA,
            pltpu.SemaphoreType.DMA,
            pltpu.SemaphoreType.DMA,
        ],
    )
    def k(g_hbm, src_hbm, dst_hbm, adt_hbm, mx_hbm, out_hbm,
          acc, srcv0, dstv0, srcv1, dstv1, gbuf0, gbuf1, adbuf0, adbuf1,
          gmaxv, zbuf, sem_g0, sem_g1, sem_a0, sem_a1, sem_s0, sem_s1,
          sem_i0, sem_i1):
        cid = lax.axis_index("c")
        sid = lax.axis_index("s")
        wid = sid * NC + cid

        nzv = ROWW // 16

        def zb(i, _):
            r = i // nzv
            col = i % nzv
            zbuf[r, pl.ds(col * 16, 16)] = jnp.zeros((16,), jnp.float32)
            return 0

        lax.fori_loop(0, ZR * nzv, zb, 0)

        t0 = sid * RPT

        def zc(j, _):
            pltpu.sync_copy(zbuf, acc.at[pl.ds(t0 + j * ZR, ZR)])
            return 0

        lax.fori_loop(0, RPT // ZR, zc, 0)
        pltpu.sync_copy(mx_hbm.at[0], gmaxv)
        base = wid * NCH
        pltpu.sync_copy(src_hbm.at[pl.ds(base, SEGCH)], srcv0)
        pltpu.sync_copy(dst_hbm.at[pl.ds(base, SEGCH)], dstv0)
        plsc.subcore_barrier()

        gm = gmaxv[:]
        GB = [gbuf0, gbuf1]
        AB = [adbuf0, adbuf1]
        SG = [sem_g0, sem_g1]
        SA = [sem_a0, sem_a1]
        SS = [sem_s0, sem_s1]
        IS = [(srcv0, dstv0, sem_i0), (srcv1, dstv1, sem_i1)]

        def compute(gbuf, adbuf):
            def edge(e, _2):
                ev = adbuf[e, :] + gbuf[e, pl.ds(FH, 16)]
                ev = jnp.where(ev > 0, ev, ALPHA * ev)
                w = jnp.exp(ev - gm)
                gbuf[e, pl.ds(FH, 16)] = w
                if n_heads == 1:
                    for kk in range(FH // 16):
                        gbuf[e, pl.ds(kk * 16, 16)] = (
                            gbuf[e, pl.ds(kk * 16, 16)] * w)
                else:
                    for hh in range(n_heads):
                        ws = w[hh]
                        gbuf[e, pl.ds(hh * 16, 16)] = (
                            gbuf[e, pl.ds(hh * 16, 16)] * ws)
                return 0

            lax.fori_loop(0, C, edge, 0)

        def gather(sv, dv, row, bi):
            pltpu.async_copy(g_hbm.at[sv.at[row]], GB[bi], SG[bi])
            pltpu.async_copy(adt_hbm.at[dv.at[row]], AB[bi], SA[bi])

        def wait_gather(sv, dv, row, bi):
            pltpu.make_async_copy(g_hbm.at[sv.at[row]], GB[bi], SG[bi]).wait()
            pltpu.make_async_copy(adt_hbm.at[dv.at[row]], AB[bi], SA[bi]).wait()

        def wait_scatter(dv, bi):
            pltpu.make_async_copy(GB[bi], acc.at[dv.at[0]], SS[bi]).wait()

        gather(srcv0, dstv0, 0, 0)

        L = SEGCH - 1
        for s in range(NSEG):
            pe = s % 2
            po = 1 - pe
            sv, dv, _ = IS[s % 2]
            if s < NSEG - 1:
                nsv, ndv, nsem = IS[(s + 1) % 2]
            else:
                nsv = ndv = nsem = None

            def pair(g, _, sv=sv, dv=dv, pe=pe, po=po, s=s,
                     nsv=nsv, ndv=ndv, nsem=nsem):
                i0 = 2 * g
                if s == 0:
                    @pl.when(g > 0)
                    def _():
                        wait_scatter(dv, po)
                else:
                    wait_scatter(dv, po)
                if nsem is not None:
                    @pl.when(g == 0)
                    def _():
                        off = base + (s + 1) * SEGCH
                        pltpu.async_copy(
                            src_hbm.at[pl.ds(off, SEGCH)], nsv, nsem)
                        pltpu.async_copy(
                            dst_hbm.at[pl.ds(off, SEGCH)], ndv, nsem)
                gather(sv, dv, i0 + 1, po)
                wait_gather(sv, dv, i0, pe)
                compute(GB[pe], AB[pe])
                pltpu.async_copy(
                    GB[pe], acc.at[dv.at[i0]], SS[pe], add=True).wait()
                gather(sv, dv, i0 + 2, pe)
                wait_gather(sv, dv, i0 + 1, po)
                compute(GB[po], AB[po])
                pltpu.async_copy(GB[po], acc.at[dv.at[i0 + 1]], SS[po],
                                 add=True)
                return 0

            lax.fori_loop(0, (SEGCH - 1) // 2, pair, 0)

            wait_scatter(dv, po)
            if nsem is not None:
                off = base + (s + 1) * SEGCH
                pltpu.make_async_copy(
                    src_hbm.at[pl.ds(off, SEGCH)], nsv, nsem).wait()
                pltpu.make_async_copy(
                    dst_hbm.at[pl.ds(off, SEGCH)], ndv, nsem).wait()
                gather(nsv, ndv, 0, po)
            wait_gather(sv, dv, L, pe)
            compute(GB[pe], AB[pe])
            if nsem is not None:
                pltpu.async_copy(GB[pe], acc.at[dv.at[L]], SS[pe], add=True)
            else:
                pltpu.sync_copy(GB[pe], acc.at[dv.at[L]], add=True)

        plsc.subcore_barrier()
        pltpu.sync_copy(acc.at[pl.ds(t0, RPT)],
                        out_hbm.at[cid, pl.ds(t0, RPT)])

    return k


@functools.lru_cache(maxsize=None)
def _edge_kernels():
    return (_make_edge_kernel(IN_F, 144, HEADS),
            _make_edge_kernel(IN_F, 144, 1),
            _make_edge_kernel(OUT_F, 80, 1))


def kernel(x, edge_index, W_att, a_att, W_hid, a_hid, W_out, a_out):
    f32 = jnp.float32
    x = x.astype(f32)
    src = edge_index[0].reshape(NW * NCH, C)
    dst = edge_index[1].reshape(NW * NCH, C)
    grid = (N // R,)

    Wcat = jnp.transpose(W_att, (1, 0, 2)).reshape(IN_F, HEADS * HID)
    eye8 = jnp.eye(HEADS, dtype=f32)
    Adst1 = (a_att[:, :HID][:, :, None] * eye8[:, None, :]).reshape(
        HEADS * HID, HEADS)
    Asrc1 = (a_att[:, HID:][:, :, None] * eye8[:, None, :]).reshape(
        HEADS * HID, HEADS)
    R8 = jnp.repeat(eye8, HID, axis=1)
    ones16 = jnp.ones((1, 16), f32)
    Adst2 = a_hid[:IN_F][:, None] * ones16
    Asrc2 = a_hid[IN_F:][:, None] * ones16
    Adst3 = a_out[:OUT_F][:, None] * ones16
    Asrc3 = a_out[OUT_F:][:, None] * ones16
    Rep = jnp.full((16, OUT_F), 1.0 / 16.0, f32)
    _sc1, _sc2, _sc3 = _edge_kernels()

    g1, adt1, mx1 = pl.pallas_call(
        _p1_body,
        grid=grid,
        in_specs=[_row_spec(IN_F), _full_spec(IN_F, IN_F),
                  _full_spec(IN_F, HEADS), _full_spec(IN_F, HEADS)],
        out_specs=[_row_spec(144), _row_spec(16), _MX_SPEC],
        out_shape=[jax.ShapeDtypeStruct((N, 144), f32),
                   jax.ShapeDtypeStruct((N, 16), f32),
                   jax.ShapeDtypeStruct((2, 16), f32)],
    )(x, Wcat, Adst1, Asrc1)

    part1 = _sc1(g1, src, dst, adt1, mx1)

    g2, adt2, mx2 = pl.pallas_call(
        functools.partial(_p23_body, relu=False),
        grid=grid,
        in_specs=[_full_spec(IN_F, IN_F), _full_spec(IN_F, 16),
                  _full_spec(IN_F, 16), _full_spec(8, IN_F),
                  _part_spec(0, 144), _part_spec(1, 144)],
        out_specs=[_row_spec(144), _row_spec(16), _MX_SPEC],
        out_shape=[jax.ShapeDtypeStruct((N, 144), f32),
                   jax.ShapeDtypeStruct((N, 16), f32),
                   jax.ShapeDtypeStruct((2, 16), f32)],
    )(W_hid, Asrc2, Adst2, R8, part1, part1)

    part2 = _sc2(g2, src, dst, adt2, mx2)

    g3, adt3, mx3 = pl.pallas_call(
        functools.partial(_p23_body, relu=True),
        grid=grid,
        in_specs=[_full_spec(IN_F, OUT_F), _full_spec(OUT_F, 16),
                  _full_spec(OUT_F, 16), _full_spec(8, IN_F),
                  _part_spec(0, 144), _part_spec(1, 144)],
        out_specs=[_row_spec(80), _row_spec(16), _MX_SPEC],
        out_shape=[jax.ShapeDtypeStruct((N, 80), f32),
                   jax.ShapeDtypeStruct((N, 16), f32),
                   jax.ShapeDtypeStruct((2, 16), f32)],
    )(W_out, Asrc3, Adst3, R8, part2, part2)

    part3 = _sc3(g3, src, dst, adt3, mx3)

    out = pl.pallas_call(
        _p4_body,
        grid=grid,
        in_specs=[_full_spec(16, OUT_F),
                  _part_spec(0, 80), _part_spec(1, 80)],
        out_specs=[_row_spec(OUT_F)],
        out_shape=[jax.ShapeDtypeStruct((N, OUT_F), f32)],
    )(Rep, part3, part3)[0]

    return out


# speedup vs baseline: 66.0936x; 1.1181x over previous
import functools

import jax
import jax.numpy as jnp
from jax import lax
from jax.experimental import pallas as pl
from jax.experimental.pallas import tpu as pltpu
from jax.experimental.pallas import tpu_sc as plsc

N = 10000
E = 320000
IN_F = 128
HID = 16
HEADS = 8
OUT_F = 64
ALPHA = 0.2
NEG = -1e30

NC = 2
NT = 16
NW = NC * NT
PT = E // NW
C = 40
NCH = PT // C
TROWS = 624
ZR = 8

R = 1000


def _lrelu(v):
    return jnp.where(v > 0, v, ALPHA * v)


def _p1_body(x_ref, wcat_ref, adm_ref, asm_ref, g_ref, adt_ref, mx_ref):
    h = jnp.dot(x_ref[...], wcat_ref[...], preferred_element_type=jnp.float32)
    asrc = jnp.dot(h, asm_ref[...], preferred_element_type=jnp.float32)
    adst = jnp.dot(h, adm_ref[...], preferred_element_type=jnp.float32)
    g_ref[...] = jnp.concatenate(
        [h, asrc, jnp.full((h.shape[0], 8), NEG, jnp.float32)], axis=1)
    adt_ref[...] = jnp.concatenate(
        [adst, jnp.zeros((h.shape[0], 8), jnp.float32)], axis=1)
    i = pl.program_id(0)

    @pl.when(i == 0)
    def _():
        mx_ref[...] = jnp.full((2, 16), -jnp.inf, jnp.float32)

    cd = jnp.max(adst, axis=0)
    cs = jnp.max(asrc, axis=0)
    pad = jnp.full((8,), -jnp.inf, jnp.float32)
    mx_ref[0, :] = jnp.maximum(mx_ref[0, :], jnp.concatenate([cd, pad]))
    mx_ref[1, :] = jnp.maximum(mx_ref[1, :], jnp.concatenate([cs, pad]))

    @pl.when(i == pl.num_programs(0) - 1)
    def _():
        g8 = _lrelu(mx_ref[0, 0:8] + mx_ref[1, 0:8])
        mx_ref[0, :] = jnp.concatenate([g8, jnp.zeros((8,), jnp.float32)])


def _p23_body(w_ref, asm_ref, adm_ref, r8_ref, p0_ref, p1_ref,
              g_ref, adt_ref, mx_ref, *, relu):
    s = p0_ref[...] + p1_ref[...]
    den = jnp.dot(s[:, IN_F:IN_F + 8], r8_ref[...],
                  preferred_element_type=jnp.float32) + 1e-16
    x = s[:, 0:IN_F] / den
    if relu:
        x = jnp.maximum(x, 0.0)
    else:
        x = jnp.where(x > 0, x, jnp.exp(x) - 1.0)
    h = jnp.dot(x, w_ref[...], preferred_element_type=jnp.float32)
    asrc = jnp.dot(h, asm_ref[...], preferred_element_type=jnp.float32)
    adst = jnp.dot(h, adm_ref[...], preferred_element_type=jnp.float32)
    g_ref[...] = jnp.concatenate([h, asrc], axis=1)
    adt_ref[...] = adst
    i = pl.program_id(0)

    @pl.when(i == 0)
    def _():
        mx_ref[...] = jnp.full((2, 16), -jnp.inf, jnp.float32)

    mx_ref[0, :] = jnp.maximum(mx_ref[0, :], jnp.max(adst, axis=0))
    mx_ref[1, :] = jnp.maximum(mx_ref[1, :], jnp.max(asrc, axis=0))

    @pl.when(i == pl.num_programs(0) - 1)
    def _():
        mx_ref[0, :] = _lrelu(mx_ref[0, :] + mx_ref[1, :])


def _p4_body(rep_ref, p0_ref, p1_ref, o_ref):
    s = p0_ref[...] + p1_ref[...]
    den = jnp.dot(s[:, OUT_F:OUT_F + 16], rep_ref[...],
                  preferred_element_type=jnp.float32) + 1e-16
    logits = s[:, 0:OUT_F] / den
    m = jnp.max(logits, axis=1, keepdims=True)
    z = logits - m
    lse = jnp.log(jnp.sum(jnp.exp(z), axis=1, keepdims=True))
    o_ref[...] = z - lse


def _row_spec(width):
    return pl.BlockSpec((R, width), lambda i: (i, 0))


def _full_spec(a, b):
    return pl.BlockSpec((a, b), lambda i: (0, 0))


def _part_spec(j, width):
    return pl.BlockSpec((None, R, width), lambda i, j=j: (j, i, 0))


_MX_SPEC = pl.BlockSpec((2, 16), lambda i: (0, 0))


def _make_edge_kernel(FH, ROWW, n_heads):
    mesh = plsc.VectorSubcoreMesh(core_axis_name="c", subcore_axis_name="s")

    @functools.partial(
        pl.kernel,
        out_type=jax.ShapeDtypeStruct((NC, N, ROWW), jnp.float32),
        mesh=mesh,
        compiler_params=pltpu.CompilerParams(use_tc_tiling_on_sc=False),
        scratch_types=[
            pltpu.VMEM_SHARED((N, ROWW), jnp.float32),
            pltpu.VMEM((NCH, C), jnp.int32),
            pltpu.VMEM((NCH, C), jnp.int32),
            pltpu.VMEM((C, ROWW), jnp.float32),
            pltpu.VMEM((C, ROWW), jnp.float32),
            pltpu.VMEM((C, ROWW), jnp.float32),
            pltpu.VMEM((C, 16), jnp.float32),
            pltpu.VMEM((C, 16), jnp.float32),
            pltpu.VMEM((C, 16), jnp.float32),
            pltpu.VMEM((16,), jnp.float32),
            pltpu.VMEM((ZR, ROWW), jnp.float32),
            pltpu.SemaphoreType.DMA,
            pltpu.SemaphoreType.DMA,
            pltpu.SemaphoreType.DMA,
            pltpu.SemaphoreType.DMA,
            pltpu.SemaphoreType.DMA,
            pltpu.SemaphoreType.DMA,
            pltpu.SemaphoreType.DMA,
            pltpu.SemaphoreType.DMA,
            pltpu.SemaphoreType.DMA,
        ],
    )
    def k(g_hbm, src_hbm, dst_hbm, adt_hbm, mx_hbm, out_hbm,
          acc, srcv, dstv, gb0, gb1, gb2, ab0, ab1, ab2, gmaxv, zbuf,
          sg0, sg1, sg2, sa0, sa1, sa2, ss0, ss1, ss2):
        cid = lax.axis_index("c")
        sid = lax.axis_index("s")
        wid = sid * NC + cid

        nzv = ROWW // 16

        def zb(i, _):
            r = i // nzv
            col = i % nzv
            zbuf[r, pl.ds(col * 16, 16)] = jnp.zeros((16,), jnp.float32)
            return 0

        lax.fori_loop(0, ZR * nzv, zb, 0)

        t0 = sid * TROWS

        def zc(j, _):
            pltpu.sync_copy(zbuf, acc.at[pl.ds(t0 + j * ZR, ZR)])
            return 0

        lax.fori_loop(0, TROWS // ZR, zc, 0)

        @pl.when(sid == NT - 1)
        def _():
            def zc2(j, _):
                pltpu.sync_copy(zbuf, acc.at[pl.ds(NT * TROWS + j * ZR, ZR)])
                return 0

            lax.fori_loop(0, (N - NT * TROWS) // ZR, zc2, 0)

        pltpu.sync_copy(mx_hbm.at[0], gmaxv)
        base = wid * NCH
        pltpu.sync_copy(src_hbm.at[pl.ds(base, NCH)], srcv)
        pltpu.sync_copy(dst_hbm.at[pl.ds(base, NCH)], dstv)
        plsc.subcore_barrier()

        gm = gmaxv[:]
        GB = [gb0, gb1, gb2]
        AB = [ab0, ab1, ab2]
        SG = [sg0, sg1, sg2]
        SA = [sa0, sa1, sa2]
        SS = [ss0, ss1, ss2]

        def compute(gbuf, adbuf):
            def edge4(q, _2):
                for u in range(4):
                    e = 4 * q + u
                    ev = adbuf[e, :] + gbuf[e, pl.ds(FH, 16)]
                    ev = jnp.where(ev > 0, ev, ALPHA * ev)
                    w = jnp.exp(ev - gm)
                    gbuf[e, pl.ds(FH, 16)] = w
                    if n_heads == 1:
                        for kk in range(FH // 16):
                            gbuf[e, pl.ds(kk * 16, 16)] = (
                                gbuf[e, pl.ds(kk * 16, 16)] * w)
                    else:
                        for hh in range(n_heads):
                            ws = w[hh]
                            gbuf[e, pl.ds(hh * 16, 16)] = (
                                gbuf[e, pl.ds(hh * 16, 16)] * ws)
                return 0

            lax.fori_loop(0, C // 4, edge4, 0)

        def gather(c, b):
            pltpu.async_copy(g_hbm.at[srcv.at[c]], GB[b], SG[b])
            pltpu.async_copy(adt_hbm.at[dstv.at[c]], AB[b], SA[b])

        def wait_gather(c, b):
            pltpu.make_async_copy(g_hbm.at[srcv.at[c]], GB[b], SG[b]).wait()
            pltpu.make_async_copy(adt_hbm.at[dstv.at[c]], AB[b], SA[b]).wait()

        def wait_scat(b):
            pltpu.make_async_copy(GB[b], acc.at[dstv.at[0]], SS[b]).wait()

        gather(0, 0)
        gather(1, 1)

        def body(g, _):
            for kk in range(3):
                c = 3 * g + kk
                wait_gather(c, kk)
                compute(GB[kk], AB[kk])

                @pl.when(c >= 1)
                def _():
                    wait_scat((kk + 2) % 3)

                @pl.when(c + 2 < NCH)
                def _():
                    gather(c + 2, (kk + 2) % 3)

                pltpu.async_copy(GB[kk], acc.at[dstv.at[c]], SS[kk],
                                 add=True)
            return 0

        lax.fori_loop(0, (NCH - 1) // 3, body, 0)

        wait_gather(NCH - 1, 0)
        compute(GB[0], AB[0])
        wait_scat(2)
        pltpu.sync_copy(GB[0], acc.at[dstv.at[NCH - 1]], add=True)

        plsc.subcore_barrier()
        pltpu.sync_copy(acc.at[pl.ds(t0, TROWS)],
                        out_hbm.at[cid, pl.ds(t0, TROWS)])

        @pl.when(sid == NT - 1)
        def _():
            pltpu.sync_copy(
                acc.at[pl.ds(NT * TROWS, N - NT * TROWS)],
                out_hbm.at[cid, pl.ds(NT * TROWS, N - NT * TROWS)])

    return k


@functools.lru_cache(maxsize=None)
def _edge_kernels():
    return (_make_edge_kernel(IN_F, 144, HEADS),
            _make_edge_kernel(IN_F, 144, 1),
            _make_edge_kernel(OUT_F, 80, 1))


def kernel(x, edge_index, W_att, a_att, W_hid, a_hid, W_out, a_out):
    f32 = jnp.float32
    x = x.astype(f32)
    src = edge_index[0].reshape(NW * NCH, C)
    dst = edge_index[1].reshape(NW * NCH, C)
    grid = (N // R,)

    Wcat = jnp.transpose(W_att, (1, 0, 2)).reshape(IN_F, HEADS * HID)
    eye8 = jnp.eye(HEADS, dtype=f32)
    Adst1 = (a_att[:, :HID][:, :, None] * eye8[:, None, :]).reshape(
        HEADS * HID, HEADS)
    Asrc1 = (a_att[:, HID:][:, :, None] * eye8[:, None, :]).reshape(
        HEADS * HID, HEADS)
    R8 = jnp.repeat(eye8, HID, axis=1)
    ones16 = jnp.ones((1, 16), f32)
    Adst2 = a_hid[:IN_F][:, None] * ones16
    Asrc2 = a_hid[IN_F:][:, None] * ones16
    Adst3 = a_out[:OUT_F][:, None] * ones16
    Asrc3 = a_out[OUT_F:][:, None] * ones16
    Rep = jnp.full((16, OUT_F), 1.0 / 16.0, f32)
    _sc1, _sc2, _sc3 = _edge_kernels()

    g1, adt1, mx1 = pl.pallas_call(
        _p1_body,
        grid=grid,
        in_specs=[_row_spec(IN_F), _full_spec(IN_F, IN_F),
                  _full_spec(IN_F, HEADS), _full_spec(IN_F, HEADS)],
        out_specs=[_row_spec(144), _row_spec(16), _MX_SPEC],
        out_shape=[jax.ShapeDtypeStruct((N, 144), f32),
                   jax.ShapeDtypeStruct((N, 16), f32),
                   jax.ShapeDtypeStruct((2, 16), f32)],
    )(x, Wcat, Adst1, Asrc1)

    part1 = _sc1(g1, src, dst, adt1, mx1)

    g2, adt2, mx2 = pl.pallas_call(
        functools.partial(_p23_body, relu=False),
        grid=grid,
        in_specs=[_full_spec(IN_F, IN_F), _full_spec(IN_F, 16),
                  _full_spec(IN_F, 16), _full_spec(8, IN_F),
                  _part_spec(0, 144), _part_spec(1, 144)],
        out_specs=[_row_spec(144), _row_spec(16), _MX_SPEC],
        out_shape=[jax.ShapeDtypeStruct((N, 144), f32),
                   jax.ShapeDtypeStruct((N, 16), f32),
                   jax.ShapeDtypeStruct((2, 16), f32)],
    )(W_hid, Asrc2, Adst2, R8, part1, part1)

    part2 = _sc2(g2, src, dst, adt2, mx2)

    g3, adt3, mx3 = pl.pallas_call(
        functools.partial(_p23_body, relu=True),
        grid=grid,
        in_specs=[_full_spec(IN_F, OUT_F), _full_spec(OUT_F, 16),
                  _full_spec(OUT_F, 16), _full_spec(8, IN_F),
                  _part_spec(0, 144), _part_spec(1, 144)],
        out_specs=[_row_spec(80), _row_spec(16), _MX_SPEC],
        out_shape=[jax.ShapeDtypeStruct((N, 80), f32),
                   jax.ShapeDtypeStruct((N, 16), f32),
                   jax.ShapeDtypeStruct((2, 16), f32)],
    )(W_out, Asrc3, Adst3, R8, part2, part2)

    part3 = _sc3(g3, src, dst, adt3, mx3)

    out = pl.pallas_call(
        _p4_body,
        grid=grid,
        in_specs=[_full_spec(16, OUT_F),
                  _part_spec(0, 80), _part_spec(1, 80)],
        out_specs=[_row_spec(OUT_F)],
        out_shape=[jax.ShapeDtypeStruct((N, OUT_F), f32)],
    )(Rep, part3, part3)[0]

    return out


# speedup vs baseline: 77.9818x; 1.1799x over previous
import functools

import jax
import jax.numpy as jnp
from jax import lax
from jax.experimental import pallas as pl
from jax.experimental.pallas import tpu as pltpu
from jax.experimental.pallas import tpu_sc as plsc

N = 10000
E = 320000
IN_F = 128
HID = 16
HEADS = 8
OUT_F = 64
ALPHA = 0.2
NEG = -1e30

NC = 2
NT = 16
NW = NC * NT
PT = E // NW
C = 40
NCH = PT // C
TROWS = 624
ZR = 8

R = 1000


def _lrelu(v):
    return jnp.where(v > 0, v, ALPHA * v)


def _pack(hb, aux):
    pk = jax.lax.bitcast_convert_type(
        hb.reshape(hb.shape[0], hb.shape[1] // 2, 2), jnp.float32)
    return jnp.concatenate([pk, aux], axis=1)


def _p1_body(x_ref, wcat_ref, adm_ref, asm_ref, hb_ref, aux_ref, adt_ref, mx_ref):
    h = jnp.dot(x_ref[...], wcat_ref[...], preferred_element_type=jnp.float32)
    asrc = jnp.dot(h, asm_ref[...], preferred_element_type=jnp.float32)
    adst = jnp.dot(h, adm_ref[...], preferred_element_type=jnp.float32)
    hb_ref[...] = h.astype(jnp.bfloat16)
    aux_ref[...] = jnp.concatenate(
        [asrc, jnp.full((h.shape[0], 8), NEG, jnp.float32)], axis=1)
    adt_ref[...] = jnp.concatenate(
        [adst, jnp.zeros((h.shape[0], 8), jnp.float32)], axis=1)
    i = pl.program_id(0)

    @pl.when(i == 0)
    def _():
        mx_ref[...] = jnp.full((2, 16), -jnp.inf, jnp.float32)

    cd = jnp.max(adst, axis=0)
    cs = jnp.max(asrc, axis=0)
    pad = jnp.full((8,), -jnp.inf, jnp.float32)
    mx_ref[0, :] = jnp.maximum(mx_ref[0, :], jnp.concatenate([cd, pad]))
    mx_ref[1, :] = jnp.maximum(mx_ref[1, :], jnp.concatenate([cs, pad]))

    @pl.when(i == pl.num_programs(0) - 1)
    def _():
        g8 = _lrelu(mx_ref[0, 0:8] + mx_ref[1, 0:8])
        mx_ref[0, :] = jnp.concatenate([g8, jnp.zeros((8,), jnp.float32)])


def _p23_body(w_ref, asm_ref, adm_ref, r8_ref, p0_ref, p1_ref,
              hb_ref, aux_ref, adt_ref, mx_ref, *, relu):
    s = p0_ref[...] + p1_ref[...]
    den = jnp.dot(s[:, IN_F:IN_F + 8], r8_ref[...],
                  preferred_element_type=jnp.float32) + 1e-16
    x = s[:, 0:IN_F] / den
    if relu:
        x = jnp.maximum(x, 0.0)
    else:
        x = jnp.where(x > 0, x, jnp.exp(x) - 1.0)
    h = jnp.dot(x, w_ref[...], preferred_element_type=jnp.float32)
    asrc = jnp.dot(h, asm_ref[...], preferred_element_type=jnp.float32)
    adst = jnp.dot(h, adm_ref[...], preferred_element_type=jnp.float32)
    hb_ref[...] = h.astype(jnp.bfloat16)
    aux_ref[...] = asrc
    adt_ref[...] = adst
    i = pl.program_id(0)

    @pl.when(i == 0)
    def _():
        mx_ref[...] = jnp.full((2, 16), -jnp.inf, jnp.float32)

    mx_ref[0, :] = jnp.maximum(mx_ref[0, :], jnp.max(adst, axis=0))
    mx_ref[1, :] = jnp.maximum(mx_ref[1, :], jnp.max(asrc, axis=0))

    @pl.when(i == pl.num_programs(0) - 1)
    def _():
        mx_ref[0, :] = _lrelu(mx_ref[0, :] + mx_ref[1, :])


def _p4_body(rep_ref, pm_ref, p0_ref, p1_ref, o_ref):
    s = p0_ref[...] + p1_ref[...]
    den = jnp.dot(s[:, OUT_F:OUT_F + 16], rep_ref[...],
                  preferred_element_type=jnp.float32) + 1e-16
    logits = s[:, 0:OUT_F] / den
    m = jnp.max(logits, axis=1, keepdims=True)
    z = logits - m
    lse = jnp.log(jnp.sum(jnp.exp(z), axis=1, keepdims=True))
    o_ref[...] = jnp.dot(z - lse, pm_ref[...],
                         preferred_element_type=jnp.float32)


def _row_spec(width):
    return pl.BlockSpec((R, width), lambda i: (i, 0))


def _full_spec(a, b):
    return pl.BlockSpec((a, b), lambda i: (0, 0))


def _part_spec(j, width):
    return pl.BlockSpec((None, R, width), lambda i, j=j: (j, i, 0))


_MX_SPEC = pl.BlockSpec((2, 16), lambda i: (0, 0))


def _make_edge_kernel(FH, ROWW, n_heads):
    mesh = plsc.VectorSubcoreMesh(core_axis_name="c", subcore_axis_name="s")
    GW = FH // 2 + 16

    @functools.partial(
        pl.kernel,
        out_type=jax.ShapeDtypeStruct((NC, N, ROWW), jnp.float32),
        mesh=mesh,
        compiler_params=pltpu.CompilerParams(use_tc_tiling_on_sc=False,
                                             needs_layout_passes=False),
        scratch_types=[
            pltpu.VMEM_SHARED((N, ROWW), jnp.float32),
            pltpu.VMEM((NCH, C), jnp.int32),
            pltpu.VMEM((C,), jnp.int32),
            pltpu.VMEM((C,), jnp.int32),
            pltpu.VMEM((C,), jnp.int32),
            pltpu.VMEM((C,), jnp.int32),
            pltpu.VMEM((C,), jnp.int32),
            pltpu.VMEM((C,), jnp.int32),
            pltpu.VMEM((C, GW), jnp.float32),
            pltpu.VMEM((C, GW), jnp.float32),
            pltpu.VMEM((C, GW), jnp.float32),
            pltpu.VMEM((C, ROWW), jnp.float32),
            pltpu.VMEM((C, ROWW), jnp.float32),
            pltpu.VMEM((C, ROWW), jnp.float32),
            pltpu.VMEM((C, 16), jnp.float32),
            pltpu.VMEM((C, 16), jnp.float32),
            pltpu.VMEM((C, 16), jnp.float32),
            pltpu.VMEM((16,), jnp.float32),
            pltpu.VMEM((ZR, ROWW), jnp.float32),
            pltpu.SemaphoreType.DMA,
            pltpu.SemaphoreType.DMA,
            pltpu.SemaphoreType.DMA,
            pltpu.SemaphoreType.DMA,
            pltpu.SemaphoreType.DMA,
            pltpu.SemaphoreType.DMA,
            pltpu.SemaphoreType.DMA,
            pltpu.SemaphoreType.DMA,
            pltpu.SemaphoreType.DMA,
        ],
    )
    def k(g_hbm, pk_hbm, adt_hbm, mx_hbm, out_hbm,
          acc, pkidx, sb0i, sb1i, sb2i, db0, db1, db2,
          gb0, gb1, gb2, st0, st1, st2, ab0, ab1, ab2, gmaxv, zbuf,
          sg0, sg1, sg2, sa0, sa1, sa2, ss0, ss1, ss2):
        cid = lax.axis_index("c")
        sid = lax.axis_index("s")
        wid = sid * NC + cid

        nzv = ROWW // 16

        def zb(i, _):
            r = i // nzv
            col = i % nzv
            zbuf[r, pl.ds(col * 16, 16)] = jnp.zeros((16,), jnp.float32)
            return 0

        lax.fori_loop(0, ZR * nzv, zb, 0)

        t0 = sid * TROWS

        def zc(j, _):
            pltpu.async_copy(zbuf, acc.at[pl.ds(t0 + j * ZR, ZR)], sg0)
            return 0

        lax.fori_loop(0, TROWS // ZR, zc, 0)

        @pl.when(sid == NT - 1)
        def _():
            def zc2(j, _):
                pltpu.async_copy(zbuf, acc.at[pl.ds(NT * TROWS + j * ZR, ZR)],
                                 sg0)
                return 0

            lax.fori_loop(0, (N - NT * TROWS) // ZR, zc2, 0)

        base = wid * NCH
        pltpu.async_copy(mx_hbm.at[0], gmaxv, sa0)
        pltpu.async_copy(pk_hbm.at[pl.ds(base, NCH)], pkidx, sa1)

        def zw(j, _):
            pltpu.make_async_copy(zbuf, acc.at[pl.ds(t0, ZR)], sg0).wait()
            return 0

        lax.fori_loop(0, TROWS // ZR, zw, 0)

        @pl.when(sid == NT - 1)
        def _():
            def zw2(j, _):
                pltpu.make_async_copy(zbuf, acc.at[pl.ds(t0, ZR)], sg0).wait()
                return 0

            lax.fori_loop(0, (N - NT * TROWS) // ZR, zw2, 0)

        pltpu.make_async_copy(mx_hbm.at[0], gmaxv, sa0).wait()
        pltpu.make_async_copy(pk_hbm.at[pl.ds(base, NCH)], pkidx, sa1).wait()
        plsc.subcore_barrier()

        gm = gmaxv[:]
        SRCB = [sb0i, sb1i, sb2i]
        DSTB = [db0, db1, db2]
        GB = [gb0, gb1, gb2]
        SB = [st0, st1, st2]
        AB = [ab0, ab1, ab2]
        SG = [sg0, sg1, sg2]
        SA = [sa0, sa1, sa2]
        SS = [ss0, ss1, ss2]

        iot = lax.iota(jnp.int32, 16)
        pat_e = (iot * 2) % 8
        pat_o = pat_e + 1
        FH2 = FH // 2
        NV = FH // 32

        def bcast(w, pat):
            return lax.gather(
                w, pat[:, None],
                lax.GatherDimensionNumbers(
                    offset_dims=(), collapsed_slice_dims=(0,),
                    start_index_map=(0,)),
                (1,),
                mode=lax.GatherScatterMode.PROMISE_IN_BOUNDS)

        def unpack_idx(c, b):
            for off in (0, 16, C - 16):
                pk = pkidx[c, pl.ds(off, 16)]
                SRCB[b][pl.ds(off, 16)] = pk & 16383
                DSTB[b][pl.ds(off, 16)] = pk >> 14

        def compute(gbuf, adbuf, sbuf):
            def edge4(q, _2):
                ws = []
                for u in range(4):
                    e = 4 * q + u
                    ev = adbuf[e, :] + gbuf[e, pl.ds(FH2, 16)]
                    ev = jnp.where(ev > 0, ev, ALPHA * ev)
                    ws.append(jnp.exp(ev - gm))
                pks = []
                for u in range(4):
                    e = 4 * q + u
                    pks.append([gbuf[e, pl.ds(kk * 16, 16)]
                                for kk in range(NV)])
                outs = []
                for u in range(4):
                    if n_heads == 1:
                        wde = wdo = ws[u]
                    else:
                        wde = bcast(ws[u], pat_e)
                        wdo = bcast(ws[u], pat_o)
                    row = []
                    for kk in range(NV):
                        bb = plsc.bitcast(pks[u][kk], jnp.bfloat16)
                        a, b = plsc.unpack(
                            bb, format=plsc.PackFormat.INTERLEAVED)
                        row.append((a * wde, b * wdo))
                    outs.append(row)
                for u in range(4):
                    e = 4 * q + u
                    sbuf[e, pl.ds(FH, 16)] = ws[u]
                    for kk in range(NV):
                        a, b = outs[u][kk]
                        sbuf[e, pl.ds(32 * kk, 16)] = a
                        sbuf[e, pl.ds(32 * kk + 16, 16)] = b
                return 0

            lax.fori_loop(0, C // 4, edge4, 0)

        def gather(b):
            pltpu.async_copy(g_hbm.at[SRCB[b]], GB[b], SG[b])
            pltpu.async_copy(adt_hbm.at[DSTB[b]], AB[b], SA[b])

        def wait_gather(b):
            pltpu.make_async_copy(g_hbm.at[SRCB[b]], GB[b], SG[b]).wait()
            pltpu.make_async_copy(adt_hbm.at[DSTB[b]], AB[b], SA[b]).wait()

        def wait_scat(b):
            pltpu.make_async_copy(SB[b], acc.at[DSTB[b]], SS[b]).wait()

        unpack_idx(0, 0)
        unpack_idx(1, 1)
        gather(0)
        gather(1)

        def body(g, _):
            for kk in range(3):
                c = 3 * g + kk
                wait_gather(kk)
                compute(GB[kk], AB[kk], SB[kk])

                @pl.when(c >= 1)
                def _():
                    wait_scat((kk + 2) % 3)

                @pl.when(c + 2 < NCH)
                def _():
                    unpack_idx(c + 2, (kk + 2) % 3)
                    gather((kk + 2) % 3)

                pltpu.async_copy(SB[kk], acc.at[DSTB[kk]], SS[kk],
                                 add=True)
            return 0

        lax.fori_loop(0, (NCH - 1) // 3, body, 0)

        wait_gather(0)
        compute(GB[0], AB[0], SB[0])
        wait_scat(2)
        pltpu.sync_copy(SB[0], acc.at[DSTB[0]], add=True)

        plsc.subcore_barrier()
        pltpu.sync_copy(acc.at[pl.ds(t0, TROWS)],
                        out_hbm.at[cid, pl.ds(t0, TROWS)])

        @pl.when(sid == NT - 1)
        def _():
            pltpu.sync_copy(
                acc.at[pl.ds(NT * TROWS, N - NT * TROWS)],
                out_hbm.at[cid, pl.ds(NT * TROWS, N - NT * TROWS)])

    return k


@functools.lru_cache(maxsize=None)
def _edge_kernels():
    return (_make_edge_kernel(IN_F, 144, HEADS),
            _make_edge_kernel(IN_F, 144, 1),
            _make_edge_kernel(OUT_F, 80, 1))


def kernel(x, edge_index, W_att, a_att, W_hid, a_hid, W_out, a_out):
    f32 = jnp.float32
    x = x.astype(f32)
    pkidx = (edge_index[0] + edge_index[1] * 16384).reshape(NW * NCH, C)
    grid = (N // R,)

    Wcat = jnp.transpose(W_att, (1, 0, 2)).reshape(IN_F, HEADS * HID)
    perm = (jnp.arange(IN_F) % HEADS) * HID + jnp.arange(IN_F) // HEADS
    Wcat = Wcat[:, perm]
    eye8 = jnp.eye(HEADS, dtype=f32)
    Adst1 = (a_att[:, :HID][:, :, None] * eye8[:, None, :]).reshape(
        HEADS * HID, HEADS)
    Asrc1 = (a_att[:, HID:][:, :, None] * eye8[:, None, :]).reshape(
        HEADS * HID, HEADS)
    Adst1 = Adst1[perm, :]
    Asrc1 = Asrc1[perm, :]
    R8 = jnp.repeat(eye8, HID, axis=1)
    p128 = jnp.arange(IN_F)
    it1 = 32 * (p128 // 32) + 2 * ((p128 % 32) % 16) + ((p128 % 32) // 16)
    p64 = jnp.arange(OUT_F)
    it3 = 32 * (p64 // 32) + 2 * ((p64 % 32) % 16) + ((p64 % 32) // 16)
    perm_full1 = perm[it1]
    R8t = R8[:, perm_full1]
    W_hid_p = W_hid[perm_full1, :]
    W_out_p = W_out[it1, :]
    Rden = jnp.full((8, IN_F), 1.0 / 8.0, f32)
    Pm3 = jnp.zeros((OUT_F, OUT_F), f32).at[p64, it3].set(1.0)
    ones16 = jnp.ones((1, 16), f32)
    Adst2 = a_hid[:IN_F][:, None] * ones16
    Asrc2 = a_hid[IN_F:][:, None] * ones16
    Adst3 = a_out[:OUT_F][:, None] * ones16
    Asrc3 = a_out[OUT_F:][:, None] * ones16
    Rep = jnp.full((16, OUT_F), 1.0 / 16.0, f32)
    _sc1, _sc2, _sc3 = _edge_kernels()

    g1, aux1, adt1, mx1 = pl.pallas_call(
        _p1_body,
        grid=grid,
        in_specs=[_row_spec(IN_F), _full_spec(IN_F, IN_F),
                  _full_spec(IN_F, HEADS), _full_spec(IN_F, HEADS)],
        out_specs=[_row_spec(IN_F), _row_spec(16), _row_spec(16), _MX_SPEC],
        out_shape=[jax.ShapeDtypeStruct((N, IN_F), jnp.bfloat16),
                   jax.ShapeDtypeStruct((N, 16), f32),
                   jax.ShapeDtypeStruct((N, 16), f32),
                   jax.ShapeDtypeStruct((2, 16), f32)],
    )(x, Wcat, Adst1, Asrc1)

    part1 = _sc1(_pack(g1, aux1), pkidx, adt1, mx1)

    g2, aux2, adt2, mx2 = pl.pallas_call(
        functools.partial(_p23_body, relu=False),
        grid=grid,
        in_specs=[_full_spec(IN_F, IN_F), _full_spec(IN_F, 16),
                  _full_spec(IN_F, 16), _full_spec(8, IN_F),
                  _part_spec(0, 144), _part_spec(1, 144)],
        out_specs=[_row_spec(IN_F), _row_spec(16), _row_spec(16), _MX_SPEC],
        out_shape=[jax.ShapeDtypeStruct((N, IN_F), jnp.bfloat16),
                   jax.ShapeDtypeStruct((N, 16), f32),
                   jax.ShapeDtypeStruct((N, 16), f32),
                   jax.ShapeDtypeStruct((2, 16), f32)],
    )(W_hid_p, Asrc2, Adst2, R8t, part1, part1)

    part2 = _sc2(_pack(g2, aux2), pkidx, adt2, mx2)

    g3, aux3, adt3, mx3 = pl.pallas_call(
        functools.partial(_p23_body, relu=True),
        grid=grid,
        in_specs=[_full_spec(IN_F, OUT_F), _full_spec(OUT_F, 16),
                  _full_spec(OUT_F, 16), _full_spec(8, IN_F),
                  _part_spec(0, 144), _part_spec(1, 144)],
        out_specs=[_row_spec(OUT_F), _row_spec(16), _row_spec(16), _MX_SPEC],
        out_shape=[jax.ShapeDtypeStruct((N, OUT_F), jnp.bfloat16),
                   jax.ShapeDtypeStruct((N, 16), f32),
                   jax.ShapeDtypeStruct((N, 16), f32),
                   jax.ShapeDtypeStruct((2, 16), f32)],
    )(W_out_p, Asrc3, Adst3, Rden, part2, part2)

    part3 = _sc3(_pack(g3, aux3), pkidx, adt3, mx3)

    out = pl.pallas_call(
        _p4_body,
        grid=grid,
        in_specs=[_full_spec(16, OUT_F), _full_spec(OUT_F, OUT_F),
                  _part_spec(0, 80), _part_spec(1, 80)],
        out_specs=[_row_spec(OUT_F)],
        out_shape=[jax.ShapeDtypeStruct((N, OUT_F), f32)],
    )(Rep, Pm3, part3, part3)[0]

    return out


# speedup vs baseline: 83.7178x; 1.0736x over previous
import functools

import jax
import jax.numpy as jnp
from jax import lax
from jax.experimental import pallas as pl
from jax.experimental.pallas import tpu as pltpu
from jax.experimental.pallas import tpu_sc as plsc

N = 10000
E = 320000
IN_F = 128
HID = 16
HEADS = 8
OUT_F = 64
ALPHA = 0.2
NEG = -1e30

NC = 2
NT = 16
NW = NC * NT
PT = E // NW
C = 40
NCH = PT // C
TROWS = 624
ZR = 8

R = 1000


def _lrelu(v):
    return jnp.where(v > 0, v, ALPHA * v)


def _p1_body(x_ref, wcat_ref, adm_ref, asm_ref, g_ref, adt_ref, mx_ref):
    h = jnp.dot(x_ref[...], wcat_ref[...], preferred_element_type=jnp.float32)
    asrc = jnp.dot(h, asm_ref[...], preferred_element_type=jnp.float32)
    adst = jnp.dot(h, adm_ref[...], preferred_element_type=jnp.float32)
    g_ref[...] = jnp.concatenate(
        [h, asrc, jnp.full((h.shape[0], 8), NEG, jnp.float32)], axis=1)
    adt_ref[...] = jnp.concatenate(
        [adst, jnp.zeros((h.shape[0], 8), jnp.float32)], axis=1)
    i = pl.program_id(0)

    @pl.when(i == 0)
    def _():
        mx_ref[...] = jnp.full((2, 16), -jnp.inf, jnp.float32)

    cd = jnp.max(adst, axis=0)
    cs = jnp.max(asrc, axis=0)
    pad = jnp.full((8,), -jnp.inf, jnp.float32)
    mx_ref[0, :] = jnp.maximum(mx_ref[0, :], jnp.concatenate([cd, pad]))
    mx_ref[1, :] = jnp.maximum(mx_ref[1, :], jnp.concatenate([cs, pad]))

    @pl.when(i == pl.num_programs(0) - 1)
    def _():
        g8 = _lrelu(mx_ref[0, 0:8] + mx_ref[1, 0:8])
        mx_ref[0, :] = jnp.concatenate([g8, jnp.zeros((8,), jnp.float32)])


def _p23_body(w_ref, asm_ref, adm_ref, r8_ref, p0_ref, p1_ref,
              g_ref, adt_ref, mx_ref, *, relu):
    s = p0_ref[...] + p1_ref[...]
    den = jnp.dot(s[:, IN_F:IN_F + 8], r8_ref[...],
                  preferred_element_type=jnp.float32) + 1e-16
    x = s[:, 0:IN_F] / den
    if relu:
        x = jnp.maximum(x, 0.0)
    else:
        x = jnp.where(x > 0, x, jnp.exp(x) - 1.0)
    h = jnp.dot(x, w_ref[...], preferred_element_type=jnp.float32)
    asrc = jnp.dot(h, asm_ref[...], preferred_element_type=jnp.float32)
    adst = jnp.dot(h, adm_ref[...], preferred_element_type=jnp.float32)
    g_ref[...] = jnp.concatenate([h, asrc], axis=1)
    adt_ref[...] = adst
    i = pl.program_id(0)

    @pl.when(i == 0)
    def _():
        mx_ref[...] = jnp.full((2, 16), -jnp.inf, jnp.float32)

    mx_ref[0, :] = jnp.maximum(mx_ref[0, :], jnp.max(adst, axis=0))
    mx_ref[1, :] = jnp.maximum(mx_ref[1, :], jnp.max(asrc, axis=0))

    @pl.when(i == pl.num_programs(0) - 1)
    def _():
        mx_ref[0, :] = _lrelu(mx_ref[0, :] + mx_ref[1, :])


def _p4_body(rep_ref, p0_ref, p1_ref, o_ref):
    s = p0_ref[...] + p1_ref[...]
    den = jnp.dot(s[:, OUT_F:OUT_F + 16], rep_ref[...],
                  preferred_element_type=jnp.float32) + 1e-16
    logits = s[:, 0:OUT_F] / den
    m = jnp.max(logits, axis=1, keepdims=True)
    z = logits - m
    lse = jnp.log(jnp.sum(jnp.exp(z), axis=1, keepdims=True))
    o_ref[...] = z - lse


def _row_spec(width):
    return pl.BlockSpec((R, width), lambda i: (i, 0))


def _full_spec(a, b):
    return pl.BlockSpec((a, b), lambda i: (0, 0))


def _part_spec(j, width):
    return pl.BlockSpec((None, R, width), lambda i, j=j: (j, i, 0))


_MX_SPEC = pl.BlockSpec((2, 16), lambda i: (0, 0))


def _make_edge_kernel(FH, ROWW, n_heads):
    mesh = plsc.VectorSubcoreMesh(core_axis_name="c", subcore_axis_name="s")

    @functools.partial(
        pl.kernel,
        out_type=jax.ShapeDtypeStruct((NC, N, ROWW), jnp.float32),
        mesh=mesh,
        compiler_params=pltpu.CompilerParams(use_tc_tiling_on_sc=False),
        scratch_types=[
            pltpu.VMEM_SHARED((N, ROWW), jnp.float32),
            pltpu.VMEM((NCH, C), jnp.int32),
            pltpu.VMEM((NCH, C), jnp.int32),
            pltpu.VMEM((C, ROWW), jnp.float32),
            pltpu.VMEM((C, ROWW), jnp.float32),
            pltpu.VMEM((C, ROWW), jnp.float32),
            pltpu.VMEM((C, 16), jnp.float32),
            pltpu.VMEM((C, 16), jnp.float32),
            pltpu.VMEM((C, 16), jnp.float32),
            pltpu.VMEM((16,), jnp.float32),
            pltpu.VMEM((ZR, ROWW), jnp.float32),
            pltpu.SemaphoreType.DMA,
            pltpu.SemaphoreType.DMA,
            pltpu.SemaphoreType.DMA,
            pltpu.SemaphoreType.DMA,
            pltpu.SemaphoreType.DMA,
            pltpu.SemaphoreType.DMA,
            pltpu.SemaphoreType.DMA,
            pltpu.SemaphoreType.DMA,
            pltpu.SemaphoreType.DMA,
        ],
    )
    def k(g_hbm, src_hbm, dst_hbm, adt_hbm, mx_hbm, out_hbm,
          acc, srcv, dstv, gb0, gb1, gb2, ab0, ab1, ab2, gmaxv, zbuf,
          sg0, sg1, sg2, sa0, sa1, sa2, ss0, ss1, ss2):
        cid = lax.axis_index("c")
        sid = lax.axis_index("s")
        wid = sid * NC + cid

        nzv = ROWW // 16

        def zb(i, _):
            r = i // nzv
            col = i % nzv
            zbuf[r, pl.ds(col * 16, 16)] = jnp.zeros((16,), jnp.float32)
            return 0

        lax.fori_loop(0, ZR * nzv, zb, 0)

        t0 = sid * TROWS

        def zc(j, _):
            pltpu.async_copy(zbuf, acc.at[pl.ds(t0 + j * ZR, ZR)], sg0)
            return 0

        lax.fori_loop(0, TROWS // ZR, zc, 0)

        @pl.when(sid == NT - 1)
        def _():
            def zc2(j, _):
                pltpu.async_copy(zbuf, acc.at[pl.ds(NT * TROWS + j * ZR, ZR)],
                                 sg0)
                return 0

            lax.fori_loop(0, (N - NT * TROWS) // ZR, zc2, 0)

        base = wid * NCH
        pltpu.async_copy(mx_hbm.at[0], gmaxv, sa0)
        pltpu.async_copy(src_hbm.at[pl.ds(base, NCH)], srcv, sa1)
        pltpu.async_copy(dst_hbm.at[pl.ds(base, NCH)], dstv, sa2)

        def zw(j, _):
            pltpu.make_async_copy(zbuf, acc.at[pl.ds(t0, ZR)], sg0).wait()
            return 0

        lax.fori_loop(0, TROWS // ZR, zw, 0)

        @pl.when(sid == NT - 1)
        def _():
            def zw2(j, _):
                pltpu.make_async_copy(zbuf, acc.at[pl.ds(t0, ZR)], sg0).wait()
                return 0

            lax.fori_loop(0, (N - NT * TROWS) // ZR, zw2, 0)

        pltpu.make_async_copy(mx_hbm.at[0], gmaxv, sa0).wait()
        pltpu.make_async_copy(src_hbm.at[pl.ds(base, NCH)], srcv, sa1).wait()
        pltpu.make_async_copy(dst_hbm.at[pl.ds(base, NCH)], dstv, sa2).wait()
        plsc.subcore_barrier()

        gm = gmaxv[:]
        GB = [gb0, gb1, gb2]
        AB = [ab0, ab1, ab2]
        SG = [sg0, sg1, sg2]
        SA = [sa0, sa1, sa2]
        SS = [ss0, ss1, ss2]

        bidx = lax.rem(lax.iota(jnp.int32, 16), jnp.int32(n_heads))

        def compute(gbuf, adbuf):
            def edge4(q, _2):
                ws = []
                for u in range(4):
                    e = 4 * q + u
                    ev = adbuf[e, :] + gbuf[e, pl.ds(FH, 16)]
                    ev = jnp.where(ev > 0, ev, ALPHA * ev)
                    ws.append(jnp.exp(ev - gm))
                fs = []
                for u in range(4):
                    e = 4 * q + u
                    fs.append([gbuf[e, pl.ds(kk * 16, 16)]
                               for kk in range(FH // 16)])
                for u in range(4):
                    if n_heads == 1:
                        wd = ws[u]
                    else:
                        wd = lax.gather(
                            ws[u], bidx[:, None],
                            lax.GatherDimensionNumbers(
                                offset_dims=(),
                                collapsed_slice_dims=(0,),
                                start_index_map=(0,)),
                            (1,),
                            mode=lax.GatherScatterMode.PROMISE_IN_BOUNDS)
                    fs[u] = [f * wd for f in fs[u]]
                for u in range(4):
                    e = 4 * q + u
                    gbuf[e, pl.ds(FH, 16)] = ws[u]
                    for kk in range(FH // 16):
                        gbuf[e, pl.ds(kk * 16, 16)] = fs[u][kk]
                return 0

            lax.fori_loop(0, C // 4, edge4, 0)

        def gather(c, b):
            pltpu.async_copy(g_hbm.at[srcv.at[c]], GB[b], SG[b])
            pltpu.async_copy(adt_hbm.at[dstv.at[c]], AB[b], SA[b])

        def wait_gather(c, b):
            pltpu.make_async_copy(g_hbm.at[srcv.at[c]], GB[b], SG[b]).wait()
            pltpu.make_async_copy(adt_hbm.at[dstv.at[c]], AB[b], SA[b]).wait()

        def wait_scat(b):
            pltpu.make_async_copy(GB[b], acc.at[dstv.at[0]], SS[b]).wait()

        gather(0, 0)
        gather(1, 1)

        def body(g, _):
            for kk in range(3):
                c = 3 * g + kk
                wait_gather(c, kk)
                compute(GB[kk], AB[kk])

                @pl.when(c >= 1)
                def _():
                    wait_scat((kk + 2) % 3)

                @pl.when(c + 2 < NCH)
                def _():
                    gather(c + 2, (kk + 2) % 3)

                pltpu.async_copy(GB[kk], acc.at[dstv.at[c]], SS[kk],
                                 add=True)
            return 0

        lax.fori_loop(0, (NCH - 1) // 3, body, 0)

        wait_gather(NCH - 1, 0)
        compute(GB[0], AB[0])
        wait_scat(2)
        pltpu.sync_copy(GB[0], acc.at[dstv.at[NCH - 1]], add=True)

        plsc.subcore_barrier()
        pltpu.sync_copy(acc.at[pl.ds(t0, TROWS)],
                        out_hbm.at[cid, pl.ds(t0, TROWS)])

        @pl.when(sid == NT - 1)
        def _():
            pltpu.sync_copy(
                acc.at[pl.ds(NT * TROWS, N - NT * TROWS)],
                out_hbm.at[cid, pl.ds(NT * TROWS, N - NT * TROWS)])

    return k


@functools.lru_cache(maxsize=None)
def _edge_kernels():
    return (_make_edge_kernel(IN_F, 144, HEADS),
            _make_edge_kernel(IN_F, 144, 1),
            _make_edge_kernel(OUT_F, 80, 1))


def kernel(x, edge_index, W_att, a_att, W_hid, a_hid, W_out, a_out):
    f32 = jnp.float32
    x = x.astype(f32)
    src = edge_index[0].reshape(NW * NCH, C)
    dst = edge_index[1].reshape(NW * NCH, C)
    grid = (N // R,)

    Wcat = jnp.transpose(W_att, (1, 0, 2)).reshape(IN_F, HEADS * HID)
    perm = (jnp.arange(IN_F) % HEADS) * HID + jnp.arange(IN_F) // HEADS
    Wcat = Wcat[:, perm]
    eye8 = jnp.eye(HEADS, dtype=f32)
    Adst1 = (a_att[:, :HID][:, :, None] * eye8[:, None, :]).reshape(
        HEADS * HID, HEADS)
    Asrc1 = (a_att[:, HID:][:, :, None] * eye8[:, None, :]).reshape(
        HEADS * HID, HEADS)
    Adst1 = Adst1[perm, :]
    Asrc1 = Asrc1[perm, :]
    R8 = jnp.repeat(eye8, HID, axis=1)
    R8t = R8[:, perm]
    W_hid_p = W_hid[perm, :]
    ones16 = jnp.ones((1, 16), f32)
    Adst2 = a_hid[:IN_F][:, None] * ones16
    Asrc2 = a_hid[IN_F:][:, None] * ones16
    Adst3 = a_out[:OUT_F][:, None] * ones16
    Asrc3 = a_out[OUT_F:][:, None] * ones16
    Rep = jnp.full((16, OUT_F), 1.0 / 16.0, f32)
    _sc1, _sc2, _sc3 = _edge_kernels()

    g1, adt1, mx1 = pl.pallas_call(
        _p1_body,
        grid=grid,
        in_specs=[_row_spec(IN_F), _full_spec(IN_F, IN_F),
                  _full_spec(IN_F, HEADS), _full_spec(IN_F, HEADS)],
        out_specs=[_row_spec(144), _row_spec(16), _MX_SPEC],
        out_shape=[jax.ShapeDtypeStruct((N, 144), f32),
                   jax.ShapeDtypeStruct((N, 16), f32),
                   jax.ShapeDtypeStruct((2, 16), f32)],
    )(x, Wcat, Adst1, Asrc1)

    part1 = _sc1(g1, src, dst, adt1, mx1)

    g2, adt2, mx2 = pl.pallas_call(
        functools.partial(_p23_body, relu=False),
        grid=grid,
        in_specs=[_full_spec(IN_F, IN_F), _full_spec(IN_F, 16),
                  _full_spec(IN_F, 16), _full_spec(8, IN_F),
                  _part_spec(0, 144), _part_spec(1, 144)],
        out_specs=[_row_spec(144), _row_spec(16), _MX_SPEC],
        out_shape=[jax.ShapeDtypeStruct((N, 144), f32),
                   jax.ShapeDtypeStruct((N, 16), f32),
                   jax.ShapeDtypeStruct((2, 16), f32)],
    )(W_hid_p, Asrc2, Adst2, R8t, part1, part1)

    part2 = _sc2(g2, src, dst, adt2, mx2)

    g3, adt3, mx3 = pl.pallas_call(
        functools.partial(_p23_body, relu=True),
        grid=grid,
        in_specs=[_full_spec(IN_F, OUT_F), _full_spec(OUT_F, 16),
                  _full_spec(OUT_F, 16), _full_spec(8, IN_F),
                  _part_spec(0, 144), _part_spec(1, 144)],
        out_specs=[_row_spec(80), _row_spec(16), _MX_SPEC],
        out_shape=[jax.ShapeDtypeStruct((N, 80), f32),
                   jax.ShapeDtypeStruct((N, 16), f32),
                   jax.ShapeDtypeStruct((2, 16), f32)],
    )(W_out, Asrc3, Adst3, R8, part2, part2)

    part3 = _sc3(g3, src, dst, adt3, mx3)

    out = pl.pallas_call(
        _p4_body,
        grid=grid,
        in_specs=[_full_spec(16, OUT_F),
                  _part_spec(0, 80), _part_spec(1, 80)],
        out_specs=[_row_spec(OUT_F)],
        out_shape=[jax.ShapeDtypeStruct((N, OUT_F), f32)],
    )(Rep, part3, part3)[0]

    return out
